# Initial kernel scaffold; baseline (speedup 1.0000x reference)
import jax
import jax.numpy as jnp
from jax.experimental import pallas as pl


def kernel(pred_cls_logit, pred_delta, img_width, real_images_width):
    raise NotImplementedError("write your pallas kernel here")


# speedup vs baseline: 771.1055x; 771.1055x over previous
import jax
import jax.numpy as jnp
from jax import lax
from jax.experimental import pallas as pl
from jax.experimental.pallas import tpu as pltpu

_FEAT_STRIDE = 16.0
_SCORE_THRESH = 0.7
_DIST_THRESH = 16.0
_MAX_OUT = 50
_OUT_PAD = 64


def _nms_body(score_ref, d0_ref, d1_ref, off_ref, maxw_ref,
              p0_ref, p1_ref, s_ref, v_ref, cls_ref,
              act_scr, m_scr, pa_scr, pb_scr):
    B, FW = score_ref.shape
    off = off_ref[...]
    maxw = maxw_ref[...]

    col_i = lax.broadcasted_iota(jnp.int32, (B, FW), 1)
    center = (col_i.astype(jnp.float32) + 0.5) * _FEAT_STRIDE + off

    def _clamp(p):
        p = jnp.where(p < 0.0, 0.0, p)
        p = jnp.where(p > maxw, maxw, p)
        return p

    p0 = _clamp(d0_ref[...] * _FEAT_STRIDE + center)
    p1 = _clamp(d1_ref[...] * _FEAT_STRIDE + center)
    pa_scr[...] = p0
    pb_scr[...] = p1
    m_scr[...] = (p0 + p1) * 0.5
    act_scr[...] = jnp.ones((B, FW), jnp.float32)
    cls_ref[...] = jnp.zeros((B, FW), jnp.float32)
    zeros_pad = jnp.zeros((B, _OUT_PAD), jnp.float32)
    p0_ref[...] = zeros_pad
    p1_ref[...] = zeros_pad
    s_ref[...] = zeros_pad
    v_ref[...] = zeros_pad

    lane = lax.broadcasted_iota(jnp.int32, (B, _OUT_PAD), 1)

    def round_body(r, carry):
        col = lax.broadcasted_iota(jnp.int32, (B, FW), 1)
        act = act_scr[...]
        msk = jnp.where(act > 0.0, score_ref[...], -1.0)
        mx = jnp.max(msk, axis=1, keepdims=True)
        valid = mx >= _SCORE_THRESH
        idx = jnp.max(jnp.where(msk == mx, col, -1), axis=1, keepdims=True)
        sel = col == idx
        p0k = jnp.sum(jnp.where(sel, pa_scr[...], 0.0), axis=1, keepdims=True)
        p1k = jnp.sum(jnp.where(sel, pb_scr[...], 0.0), axis=1, keepdims=True)
        mk = (p0k + p1k) * 0.5
        suppress = jnp.abs(m_scr[...] - mk) <= _DIST_THRESH
        act_scr[...] = jnp.where(suppress & valid, 0.0, act)
        xn = jnp.floor(mk / _FEAT_STRIDE).astype(jnp.int32)
        cls_ref[...] = cls_ref[...] + jnp.where((col == xn) & valid, 1.0, 0.0)
        vf = valid.astype(jnp.float32)
        here = lane == r
        p0_ref[...] = jnp.where(here, p0k * vf, p0_ref[...])
        p1_ref[...] = jnp.where(here, p1k * vf, p1_ref[...])
        s_ref[...] = jnp.where(here, mx * vf, s_ref[...])
        v_ref[...] = jnp.where(here, vf, v_ref[...])
        return carry

    lax.fori_loop(0, _MAX_OUT, round_body, 0)


@jax.jit
def _run(scores, d0, d1, off_col, maxw_col):
    B, FW = scores.shape
    return pl.pallas_call(
        _nms_body,
        out_shape=[
            jax.ShapeDtypeStruct((B, _OUT_PAD), jnp.float32),
            jax.ShapeDtypeStruct((B, _OUT_PAD), jnp.float32),
            jax.ShapeDtypeStruct((B, _OUT_PAD), jnp.float32),
            jax.ShapeDtypeStruct((B, _OUT_PAD), jnp.float32),
            jax.ShapeDtypeStruct((B, FW), jnp.float32),
        ],
        scratch_shapes=[
            pltpu.VMEM((B, FW), jnp.float32),
            pltpu.VMEM((B, FW), jnp.float32),
            pltpu.VMEM((B, FW), jnp.float32),
            pltpu.VMEM((B, FW), jnp.float32),
        ],
    )(scores, d0, d1, off_col, maxw_col)


def kernel(pred_cls_logit, pred_delta, img_width, real_images_width):
    B, FW = pred_cls_logit.shape
    scores = jax.nn.sigmoid(pred_cls_logit)
    d0 = pred_delta[..., 0]
    d1 = pred_delta[..., 1]
    off = (jnp.asarray(img_width) - FW * 16).astype(jnp.float32)
    off_col = jnp.broadcast_to(jnp.reshape(off, (1, 1)), (B, 1))
    maxw_col = (jnp.asarray(real_images_width, jnp.float32) - 1.0).reshape(B, 1)
    P0, P1, S, V, cls = _run(scores, d0, d1, off_col, maxw_col)
    P0, P1, S, V = (a[:, :_MAX_OUT] for a in (P0, P1, S, V))
    nms_positions = jnp.stack([P0, P1, V], axis=-1)
    nms_scores = jnp.stack([S, V], axis=-1)
    return nms_positions, nms_scores, cls
